# baseline (speedup 1.0000x reference)
import functools

import jax
import jax.numpy as jnp
from jax import lax
from jax.experimental import pallas as pl
from jax.experimental.pallas import tpu as pltpu
from jax.experimental.pallas import tpu_sc as plsc

NUM_CORES = 2
NUM_SUBCORES = 16
NUM_WORKERS = NUM_CORES * NUM_SUBCORES
LANES = 16

CHUNK = 64
ZROWS = 40
OWN_ROWS = 640
NBUF = 5
ACC_PAD = 40


def _sc_aggregate(x, edge5):
    n_nodes, d = x.shape
    n_rounds = edge5.shape[2]
    n_acc = n_nodes + ACC_PAD
    tail_rows = n_acc - (NUM_SUBCORES - 1) * OWN_ROWS
    full_blocks = OWN_ROWS // ZROWS
    tail_blocks = tail_rows // ZROWS
    out_tail_blocks = (n_nodes - (NUM_SUBCORES - 1) * OWN_ROWS) // ZROWS
    assert n_rounds % 2 == 0 and n_rounds >= 4

    mesh = plsc.VectorSubcoreMesh(core_axis_name="c", subcore_axis_name="s")

    @functools.partial(
        pl.kernel,
        out_type=jax.ShapeDtypeStruct((NUM_CORES * n_nodes, d), jnp.float32),
        mesh=mesh,
        scratch_types=[
            pltpu.VMEM((2, NBUF, CHUNK), jnp.int32),
            pltpu.VMEM((2, NBUF, CHUNK), jnp.int32),
            pltpu.VMEM((NBUF, CHUNK, d), jnp.float32),
            pltpu.VMEM((ZROWS, d), jnp.float32),
            pltpu.VMEM_SHARED((n_acc, d), jnp.float32),
            [pltpu.SemaphoreType.DMA] * NBUF,
            [pltpu.SemaphoreType.DMA] * 2,
            [pltpu.SemaphoreType.DMA] * NBUF,
            pltpu.SemaphoreType.DMA,
        ],
    )
    def agg_kernel(x_hbm, e_hbm, out_hbm,
                   sidx_v, didx_v, rows_v, buf_v, acc_sh, gsems, isems, ssems,
                   zsem):
        cid = lax.axis_index("c")
        sid = lax.axis_index("s")
        w = cid * NUM_SUBCORES + sid
        my_row0 = sid * OWN_ROWS
        my_blocks = jnp.where(sid == NUM_SUBCORES - 1, tail_blocks, full_blocks)

        def fire_idx(r, s):
            pltpu.async_copy(e_hbm.at[0, w, r], sidx_v.at[s], isems[s])
            pltpu.async_copy(e_hbm.at[1, w, r], didx_v.at[s], isems[s])

        def wait_idx(r, s):
            pltpu.make_async_copy(e_hbm.at[0, w, r], sidx_v.at[s],
                                  isems[s]).wait()
            pltpu.make_async_copy(e_hbm.at[1, w, r], didx_v.at[s],
                                  isems[s]).wait()

        def fire(s, b):
            pltpu.async_copy(x_hbm.at[sidx_v.at[s, b]], rows_v.at[b], gsems[b])

        def drain(s, b):
            pltpu.make_async_copy(
                x_hbm.at[sidx_v.at[s, b]], rows_v.at[b], gsems[b]).wait()

        def scat_fire(s, b):
            pltpu.async_copy(rows_v.at[b], acc_sh.at[didx_v.at[s, b]],
                             ssems[b], add=True)

        def scat_wait(s, b):
            pltpu.make_async_copy(rows_v.at[b], acc_sh.at[didx_v.at[s, b]],
                                  ssems[b]).wait()

        with jax.named_scope("sc_zero"):
            fire_idx(0, 0)
            fire_idx(1, 1)
            wait_idx(0, 0)
            for b in range(NBUF):
                fire(0, b)

            zero = jnp.zeros((LANES,), jnp.float32)

            def zrow(rr, _):
                for j in range(d // LANES):
                    buf_v[rr, pl.ds(j * LANES, LANES)] = zero
                return ()

            lax.fori_loop(0, ZROWS, zrow, ())

            def zblk(k, _):
                pltpu.async_copy(
                    buf_v, acc_sh.at[pl.ds(my_row0 + k * ZROWS, ZROWS)], zsem)
                return ()

            lax.fori_loop(0, my_blocks, zblk, ())

            def zwait(k, _):
                pltpu.make_async_copy(
                    buf_v, acc_sh.at[pl.ds(my_row0 + k * ZROWS, ZROWS)],
                    zsem).wait()
                return ()

            lax.fori_loop(0, my_blocks, zwait, ())
            plsc.subcore_barrier()

        def round_body(r, s, prefetch):
            wait_idx(r + 1, 1 - s)
            for b in range(NBUF):
                drain(s, b)
                scat_fire(s, b)
                if b > 0:
                    scat_wait(s, b - 1)
                    fire(1 - s, b - 1)
            scat_wait(s, NBUF - 1)
            fire(1 - s, NBUF - 1)
            if prefetch:
                fire_idx(r + 2, s)

        def body(t, _):
            round_body(2 * t, 0, True)
            round_body(2 * t + 1, 1, True)
            return ()

        with jax.named_scope("sc_mainloop"):
            lax.fori_loop(0, n_rounds // 2 - 1, body, ())
            round_body(n_rounds - 2, 0, False)
            for b in range(NBUF):
                drain(1, b)
                scat_fire(1, b)
            for b in range(NBUF):
                scat_wait(1, b)
            plsc.subcore_barrier()

        with jax.named_scope("sc_copyout"):
            out_blocks = jnp.where(sid == NUM_SUBCORES - 1,
                                   out_tail_blocks, full_blocks)

            def oblk(k, _):
                row0 = my_row0 + k * ZROWS
                pltpu.async_copy(
                    acc_sh.at[pl.ds(row0, ZROWS)],
                    out_hbm.at[pl.ds(cid * n_nodes + row0, ZROWS)], zsem)
                return ()

            lax.fori_loop(0, out_blocks, oblk, ())

            def owait(k, _):
                row0 = my_row0 + k * ZROWS
                pltpu.make_async_copy(
                    acc_sh.at[pl.ds(row0, ZROWS)],
                    out_hbm.at[pl.ds(cid * n_nodes + row0, ZROWS)], zsem).wait()
                return ()

            lax.fori_loop(0, out_blocks, owait, ())

    return agg_kernel(x, edge5)


def _tc_head(partials, batch3d, w_enc, b_enc, w_proj, b_proj, n_graphs):
    n_nodes = partials.shape[0] // 2
    d = partials.shape[1]
    c = w_proj.shape[1]
    bn = 1000
    n_steps = n_nodes // bn

    def body(p0_ref, p1_ref, b_ref, we_ref, be_ref, wp_ref, bp_ref,
             out_ref, g_acc):
        i = pl.program_id(0)
        agg = p0_ref[...] + p1_ref[...]
        h = jnp.dot(agg, we_ref[...], preferred_element_type=jnp.float32)
        h = jnp.maximum(h + be_ref[...], 0.0)
        seg = b_ref[0, 0, :]
        rows = lax.broadcasted_iota(jnp.int32, (n_graphs, bn), 0)
        onehot = (seg[None, :] == rows).astype(jnp.float32)
        contrib = jnp.dot(onehot, h, preferred_element_type=jnp.float32)

        @pl.when(i == 0)
        def _():
            g_acc[...] = contrib

        @pl.when(i > 0)
        def _():
            g_acc[...] = g_acc[...] + contrib

        @pl.when(i == n_steps - 1)
        def _():
            g = g_acc[...]
            nrm = jnp.sqrt(jnp.sum(g * g, axis=1, keepdims=True))
            g = g / jnp.maximum(nrm, 1e-12)
            o = jnp.dot(g, wp_ref[...], preferred_element_type=jnp.float32)
            o = o + bp_ref[...]
            nrm2 = jnp.sqrt(jnp.sum(o * o, axis=1, keepdims=True))
            out_ref[...] = o / jnp.maximum(nrm2, 1e-12)

    return pl.pallas_call(
        body,
        grid=(n_steps,),
        in_specs=[
            pl.BlockSpec((bn, d), lambda i: (i, 0)),
            pl.BlockSpec((bn, d), lambda i: (n_steps + i, 0)),
            pl.BlockSpec((1, 1, bn), lambda i: (i, 0, 0)),
            pl.BlockSpec((d, d), lambda i: (0, 0)),
            pl.BlockSpec((1, d), lambda i: (0, 0)),
            pl.BlockSpec((d, c), lambda i: (0, 0)),
            pl.BlockSpec((1, c), lambda i: (0, 0)),
        ],
        out_specs=pl.BlockSpec((n_graphs, c), lambda i: (0, 0)),
        out_shape=jax.ShapeDtypeStruct((n_graphs, c), jnp.float32),
        scratch_shapes=[pltpu.VMEM((n_graphs, d), jnp.float32)],
    )(partials, partials, batch3d, w_enc, b_enc, w_proj, b_proj)


def kernel(x, edge_index, batch, W_enc, b_enc, W_proj, b_proj):
    n_nodes, d = x.shape
    n_graphs = 64
    n_edges = edge_index.shape[1]
    round_edges = NUM_WORKERS * NBUF * CHUNK
    n_rounds = -(-n_edges // round_edges)
    if n_rounds % 2:
        n_rounds += 1
    n_pad = n_rounds * round_edges - n_edges
    if n_pad:
        lane = jnp.arange(n_pad, dtype=jnp.int32)
        pad = jnp.stack([lane % n_nodes,
                         n_nodes + lane % ACC_PAD])
        edge_index = jnp.concatenate([edge_index, pad], axis=1)
    edge5 = edge_index.reshape(2, NUM_WORKERS, n_rounds, NBUF, CHUNK)
    partials = _sc_aggregate(x, edge5)
    batch3d = batch.reshape(10, 1, n_nodes // 10)
    return _tc_head(partials, batch3d,
                    W_enc, b_enc.reshape(1, d), W_proj,
                    b_proj.reshape(1, W_proj.shape[1]), n_graphs)


# speedup vs baseline: 1.0276x; 1.0276x over previous
import functools

import jax
import jax.numpy as jnp
from jax import lax
from jax.experimental import pallas as pl
from jax.experimental.pallas import tpu as pltpu
from jax.experimental.pallas import tpu_sc as plsc

NUM_CORES = 2
NUM_SUBCORES = 16
NUM_WORKERS = NUM_CORES * NUM_SUBCORES
LANES = 16

CHUNK = 80
ZROWS = 40
OWN_ROWS = 640
NBUF = 4
ACC_PAD = 40


def _sc_aggregate(x, edge5):
    n_nodes, d = x.shape
    n_rounds = edge5.shape[2]
    n_acc = n_nodes + ACC_PAD
    tail_rows = n_acc - (NUM_SUBCORES - 1) * OWN_ROWS
    full_blocks = OWN_ROWS // ZROWS
    tail_blocks = tail_rows // ZROWS
    out_tail_blocks = (n_nodes - (NUM_SUBCORES - 1) * OWN_ROWS) // ZROWS
    assert n_rounds % 2 == 0 and n_rounds >= 4

    mesh = plsc.VectorSubcoreMesh(core_axis_name="c", subcore_axis_name="s")

    @functools.partial(
        pl.kernel,
        out_type=jax.ShapeDtypeStruct((NUM_CORES * n_nodes, d), jnp.float32),
        mesh=mesh,
        scratch_types=[
            pltpu.VMEM((2, NBUF, CHUNK), jnp.int32),
            pltpu.VMEM((2, NBUF, CHUNK), jnp.int32),
            pltpu.VMEM((NBUF, CHUNK, d), jnp.float32),
            pltpu.VMEM((ZROWS, d), jnp.float32),
            pltpu.VMEM_SHARED((n_acc, d), jnp.float32),
            [pltpu.SemaphoreType.DMA] * NBUF,
            [pltpu.SemaphoreType.DMA] * 2,
            [pltpu.SemaphoreType.DMA] * NBUF,
            pltpu.SemaphoreType.DMA,
        ],
    )
    def agg_kernel(x_hbm, e_hbm, out_hbm,
                   sidx_v, didx_v, rows_v, buf_v, acc_sh, gsems, isems, ssems,
                   zsem):
        cid = lax.axis_index("c")
        sid = lax.axis_index("s")
        w = cid * NUM_SUBCORES + sid
        my_row0 = sid * OWN_ROWS
        my_blocks = jnp.where(sid == NUM_SUBCORES - 1, tail_blocks, full_blocks)

        def fire_idx(r, s):
            pltpu.async_copy(e_hbm.at[0, w, r], sidx_v.at[s], isems[s])
            pltpu.async_copy(e_hbm.at[1, w, r], didx_v.at[s], isems[s])

        def wait_idx(r, s):
            pltpu.make_async_copy(e_hbm.at[0, w, r], sidx_v.at[s],
                                  isems[s]).wait()
            pltpu.make_async_copy(e_hbm.at[1, w, r], didx_v.at[s],
                                  isems[s]).wait()

        def fire(s, b):
            pltpu.async_copy(x_hbm.at[sidx_v.at[s, b]], rows_v.at[b], gsems[b])

        def drain(s, b):
            pltpu.make_async_copy(
                x_hbm.at[sidx_v.at[s, b]], rows_v.at[b], gsems[b]).wait()

        def scat_fire(s, b):
            pltpu.async_copy(rows_v.at[b], acc_sh.at[didx_v.at[s, b]],
                             ssems[b], add=True)

        def scat_wait(s, b):
            pltpu.make_async_copy(rows_v.at[b], acc_sh.at[didx_v.at[s, b]],
                                  ssems[b]).wait()

        with jax.named_scope("sc_zero"):
            fire_idx(0, 0)
            fire_idx(1, 1)
            wait_idx(0, 0)
            for b in range(NBUF):
                fire(0, b)

            zero = jnp.zeros((LANES,), jnp.float32)

            def zrow(rr, _):
                for j in range(d // LANES):
                    buf_v[rr, pl.ds(j * LANES, LANES)] = zero
                return ()

            lax.fori_loop(0, ZROWS, zrow, ())

            def zblk(k, _):
                pltpu.async_copy(
                    buf_v, acc_sh.at[pl.ds(my_row0 + k * ZROWS, ZROWS)], zsem)
                return ()

            lax.fori_loop(0, my_blocks, zblk, ())

            def zwait(k, _):
                pltpu.make_async_copy(
                    buf_v, acc_sh.at[pl.ds(my_row0 + k * ZROWS, ZROWS)],
                    zsem).wait()
                return ()

            lax.fori_loop(0, my_blocks, zwait, ())
            plsc.subcore_barrier()

        def round_body(r, s, prefetch):
            wait_idx(r + 1, 1 - s)
            for b in range(NBUF):
                drain(s, b)
                scat_fire(s, b)
                if b > 0:
                    scat_wait(s, b - 1)
                    fire(1 - s, b - 1)
            scat_wait(s, NBUF - 1)
            fire(1 - s, NBUF - 1)
            if prefetch:
                fire_idx(r + 2, s)

        def body(t, _):
            round_body(2 * t, 0, True)
            round_body(2 * t + 1, 1, True)
            return ()

        with jax.named_scope("sc_mainloop"):
            lax.fori_loop(0, n_rounds // 2 - 1, body, ())
            round_body(n_rounds - 2, 0, False)
            for b in range(NBUF):
                drain(1, b)
                scat_fire(1, b)
            for b in range(NBUF):
                scat_wait(1, b)
            plsc.subcore_barrier()

        with jax.named_scope("sc_copyout"):
            out_blocks = jnp.where(sid == NUM_SUBCORES - 1,
                                   out_tail_blocks, full_blocks)

            def oblk(k, _):
                row0 = my_row0 + k * ZROWS
                pltpu.async_copy(
                    acc_sh.at[pl.ds(row0, ZROWS)],
                    out_hbm.at[pl.ds(cid * n_nodes + row0, ZROWS)], zsem)
                return ()

            lax.fori_loop(0, out_blocks, oblk, ())

            def owait(k, _):
                row0 = my_row0 + k * ZROWS
                pltpu.make_async_copy(
                    acc_sh.at[pl.ds(row0, ZROWS)],
                    out_hbm.at[pl.ds(cid * n_nodes + row0, ZROWS)], zsem).wait()
                return ()

            lax.fori_loop(0, out_blocks, owait, ())

    return agg_kernel(x, edge5)


def _tc_head(partials, batch3d, w_enc, b_enc, w_proj, b_proj, n_graphs):
    n_nodes = partials.shape[0] // 2
    d = partials.shape[1]
    c = w_proj.shape[1]
    bn = 2000
    n_steps = n_nodes // bn

    def body(p0_ref, p1_ref, b_ref, we_ref, be_ref, wp_ref, bp_ref,
             out_ref, g_acc):
        i = pl.program_id(0)
        agg = p0_ref[...] + p1_ref[...]
        h = jnp.dot(agg, we_ref[...], preferred_element_type=jnp.float32)
        h = jnp.maximum(h + be_ref[...], 0.0)
        seg = b_ref[0, 0, :]
        rows = lax.broadcasted_iota(jnp.int32, (n_graphs, bn), 0)
        onehot = (seg[None, :] == rows).astype(jnp.float32)
        contrib = jnp.dot(onehot, h, preferred_element_type=jnp.float32)

        @pl.when(i == 0)
        def _():
            g_acc[...] = contrib

        @pl.when(i > 0)
        def _():
            g_acc[...] = g_acc[...] + contrib

        @pl.when(i == n_steps - 1)
        def _():
            g = g_acc[...]
            nrm = jnp.sqrt(jnp.sum(g * g, axis=1, keepdims=True))
            g = g / jnp.maximum(nrm, 1e-12)
            o = jnp.dot(g, wp_ref[...], preferred_element_type=jnp.float32)
            o = o + bp_ref[...]
            nrm2 = jnp.sqrt(jnp.sum(o * o, axis=1, keepdims=True))
            out_ref[...] = o / jnp.maximum(nrm2, 1e-12)

    return pl.pallas_call(
        body,
        grid=(n_steps,),
        in_specs=[
            pl.BlockSpec((bn, d), lambda i: (i, 0)),
            pl.BlockSpec((bn, d), lambda i: (n_steps + i, 0)),
            pl.BlockSpec((1, 1, bn), lambda i: (i, 0, 0)),
            pl.BlockSpec((d, d), lambda i: (0, 0)),
            pl.BlockSpec((1, d), lambda i: (0, 0)),
            pl.BlockSpec((d, c), lambda i: (0, 0)),
            pl.BlockSpec((1, c), lambda i: (0, 0)),
        ],
        out_specs=pl.BlockSpec((n_graphs, c), lambda i: (0, 0)),
        out_shape=jax.ShapeDtypeStruct((n_graphs, c), jnp.float32),
        scratch_shapes=[pltpu.VMEM((n_graphs, d), jnp.float32)],
    )(partials, partials, batch3d, w_enc, b_enc, w_proj, b_proj)


def kernel(x, edge_index, batch, W_enc, b_enc, W_proj, b_proj):
    n_nodes, d = x.shape
    n_graphs = 64
    n_edges = edge_index.shape[1]
    round_edges = NUM_WORKERS * NBUF * CHUNK
    n_rounds = -(-n_edges // round_edges)
    if n_rounds % 2:
        n_rounds += 1
    n_pad = n_rounds * round_edges - n_edges
    if n_pad:
        lane = jnp.arange(n_pad, dtype=jnp.int32)
        pad = jnp.stack([lane % n_nodes,
                         n_nodes + lane % ACC_PAD])
        edge_index = jnp.concatenate([edge_index, pad], axis=1)
    edge5 = edge_index.reshape(2, NUM_WORKERS, n_rounds, NBUF, CHUNK)
    partials = _sc_aggregate(x, edge5)
    batch3d = batch.reshape(5, 1, n_nodes // 5)
    return _tc_head(partials, batch3d,
                    W_enc, b_enc.reshape(1, d), W_proj,
                    b_proj.reshape(1, W_proj.shape[1]), n_graphs)


# speedup vs baseline: 1.0368x; 1.0089x over previous
import functools

import jax
import jax.numpy as jnp
from jax import lax
from jax.experimental import pallas as pl
from jax.experimental.pallas import tpu as pltpu
from jax.experimental.pallas import tpu_sc as plsc

NUM_CORES = 2
NUM_SUBCORES = 16
NUM_WORKERS = NUM_CORES * NUM_SUBCORES
LANES = 16

CHUNK = 80
ZROWS = 40
OWN_ROWS = 640
NBUF = 4
ACC_PAD = 40


def _sc_aggregate(x, edge5):
    n_nodes, d = x.shape
    n_rounds = edge5.shape[2]
    n_acc = n_nodes + ACC_PAD
    tail_rows = n_acc - (NUM_SUBCORES - 1) * OWN_ROWS
    full_blocks = OWN_ROWS // ZROWS
    tail_blocks = tail_rows // ZROWS
    out_tail_blocks = (n_nodes - (NUM_SUBCORES - 1) * OWN_ROWS) // ZROWS
    assert n_rounds % 2 == 0 and n_rounds >= 4

    mesh = plsc.VectorSubcoreMesh(core_axis_name="c", subcore_axis_name="s")

    @functools.partial(
        pl.kernel,
        out_type=jax.ShapeDtypeStruct((NUM_CORES * n_nodes, d), jnp.float32),
        mesh=mesh,
        scratch_types=[
            pltpu.VMEM((2, NBUF, CHUNK), jnp.int32),
            pltpu.VMEM((2, NBUF, CHUNK), jnp.int32),
            pltpu.VMEM((NBUF, CHUNK, d), jnp.float32),
            pltpu.VMEM((ZROWS, d), jnp.float32),
            pltpu.VMEM_SHARED((n_acc, d), jnp.float32),
            [pltpu.SemaphoreType.DMA] * NBUF,
            [pltpu.SemaphoreType.DMA] * 2,
            [pltpu.SemaphoreType.DMA] * NBUF,
            pltpu.SemaphoreType.DMA,
        ],
    )
    def agg_kernel(x_hbm, e_hbm, out_hbm,
                   sidx_v, didx_v, rows_v, buf_v, acc_sh, gsems, isems, ssems,
                   zsem):
        cid = lax.axis_index("c")
        sid = lax.axis_index("s")
        w = cid * NUM_SUBCORES + sid
        my_row0 = sid * OWN_ROWS
        my_blocks = jnp.where(sid == NUM_SUBCORES - 1, tail_blocks, full_blocks)

        def fire_idx(r, s):
            pltpu.async_copy(e_hbm.at[0, w, r], sidx_v.at[s], isems[s])
            pltpu.async_copy(e_hbm.at[1, w, r], didx_v.at[s], isems[s])

        def wait_idx(r, s):
            pltpu.make_async_copy(e_hbm.at[0, w, r], sidx_v.at[s],
                                  isems[s]).wait()
            pltpu.make_async_copy(e_hbm.at[1, w, r], didx_v.at[s],
                                  isems[s]).wait()

        def fire(s, b):
            pltpu.async_copy(x_hbm.at[sidx_v.at[s, b]], rows_v.at[b], gsems[b])

        def drain(s, b):
            pltpu.make_async_copy(
                x_hbm.at[sidx_v.at[s, b]], rows_v.at[b], gsems[b]).wait()

        def scat_fire(s, b):
            pltpu.async_copy(rows_v.at[b], acc_sh.at[didx_v.at[s, b]],
                             ssems[b], add=True)

        def scat_wait(s, b):
            pltpu.make_async_copy(rows_v.at[b], acc_sh.at[didx_v.at[s, b]],
                                  ssems[b]).wait()

        with jax.named_scope("sc_zero"):
            fire_idx(0, 0)
            fire_idx(1, 1)
            wait_idx(0, 0)
            for b in range(NBUF):
                fire(0, b)

            zero = jnp.zeros((LANES,), jnp.float32)

            def zrow(rr, _):
                for j in range(d // LANES):
                    buf_v[rr, pl.ds(j * LANES, LANES)] = zero
                return ()

            lax.fori_loop(0, ZROWS, zrow, ())

            def zblk(k, _):
                pltpu.async_copy(
                    buf_v, acc_sh.at[pl.ds(my_row0 + k * ZROWS, ZROWS)], zsem)
                return ()

            lax.fori_loop(0, my_blocks, zblk, ())

            def zwait(k, _):
                pltpu.make_async_copy(
                    buf_v, acc_sh.at[pl.ds(my_row0 + k * ZROWS, ZROWS)],
                    zsem).wait()
                return ()

            lax.fori_loop(0, my_blocks, zwait, ())
            plsc.subcore_barrier()

        def round_body(r, s, prefetch):
            wait_idx(r + 1, 1 - s)
            for b in range(NBUF):
                drain(s, b)
                scat_fire(s, b)
                if b > 0:
                    scat_wait(s, b - 1)
                    fire(1 - s, b - 1)
            scat_wait(s, NBUF - 1)
            fire(1 - s, NBUF - 1)
            if prefetch:
                fire_idx(r + 2, s)

        def body(t, _):
            round_body(2 * t, 0, True)
            round_body(2 * t + 1, 1, True)
            return ()

        with jax.named_scope("sc_mainloop"):
            lax.fori_loop(0, n_rounds // 2 - 1, body, ())
            round_body(n_rounds - 2, 0, False)
            for b in range(NBUF):
                drain(1, b)
                scat_fire(1, b)
            for b in range(NBUF):
                scat_wait(1, b)
            plsc.subcore_barrier()

        with jax.named_scope("sc_copyout"):
            out_blocks = jnp.where(sid == NUM_SUBCORES - 1,
                                   out_tail_blocks, full_blocks)

            def oblk(k, _):
                row0 = my_row0 + k * ZROWS
                pltpu.async_copy(
                    acc_sh.at[pl.ds(row0, ZROWS)],
                    out_hbm.at[pl.ds(cid * n_nodes + row0, ZROWS)], zsem)
                return ()

            lax.fori_loop(0, out_blocks, oblk, ())

            def owait(k, _):
                row0 = my_row0 + k * ZROWS
                pltpu.make_async_copy(
                    acc_sh.at[pl.ds(row0, ZROWS)],
                    out_hbm.at[pl.ds(cid * n_nodes + row0, ZROWS)], zsem).wait()
                return ()

            lax.fori_loop(0, out_blocks, owait, ())

    return agg_kernel(x, edge5)


def _tc_head(partials, batch3d, w_enc, b_enc, w_proj, b_proj, n_graphs):
    n_nodes = partials.shape[0] // 2
    d = partials.shape[1]
    c = w_proj.shape[1]
    bn = 5000
    n_steps = n_nodes // bn

    def body(p0_ref, p1_ref, b_ref, we_ref, be_ref, wp_ref, bp_ref,
             out_ref, g_acc):
        i = pl.program_id(0)
        agg = p0_ref[...] + p1_ref[...]
        h = jnp.dot(agg, we_ref[...], preferred_element_type=jnp.float32)
        h = jnp.maximum(h + be_ref[...], 0.0)
        seg = b_ref[0, 0, :]
        rows = lax.broadcasted_iota(jnp.int32, (n_graphs, bn), 0)
        onehot = (seg[None, :] == rows).astype(jnp.float32)
        contrib = jnp.dot(onehot, h, preferred_element_type=jnp.float32)

        @pl.when(i == 0)
        def _():
            g_acc[...] = contrib

        @pl.when(i > 0)
        def _():
            g_acc[...] = g_acc[...] + contrib

        @pl.when(i == n_steps - 1)
        def _():
            g = g_acc[...]
            nrm = jnp.sqrt(jnp.sum(g * g, axis=1, keepdims=True))
            g = g / jnp.maximum(nrm, 1e-12)
            o = jnp.dot(g, wp_ref[...], preferred_element_type=jnp.float32)
            o = o + bp_ref[...]
            nrm2 = jnp.sqrt(jnp.sum(o * o, axis=1, keepdims=True))
            out_ref[...] = o / jnp.maximum(nrm2, 1e-12)

    return pl.pallas_call(
        body,
        grid=(n_steps,),
        in_specs=[
            pl.BlockSpec((bn, d), lambda i: (i, 0)),
            pl.BlockSpec((bn, d), lambda i: (n_steps + i, 0)),
            pl.BlockSpec((1, 1, bn), lambda i: (i, 0, 0)),
            pl.BlockSpec((d, d), lambda i: (0, 0)),
            pl.BlockSpec((1, d), lambda i: (0, 0)),
            pl.BlockSpec((d, c), lambda i: (0, 0)),
            pl.BlockSpec((1, c), lambda i: (0, 0)),
        ],
        out_specs=pl.BlockSpec((n_graphs, c), lambda i: (0, 0)),
        out_shape=jax.ShapeDtypeStruct((n_graphs, c), jnp.float32),
        scratch_shapes=[pltpu.VMEM((n_graphs, d), jnp.float32)],
    )(partials, partials, batch3d, w_enc, b_enc, w_proj, b_proj)


def kernel(x, edge_index, batch, W_enc, b_enc, W_proj, b_proj):
    n_nodes, d = x.shape
    n_graphs = 64
    n_edges = edge_index.shape[1]
    round_edges = NUM_WORKERS * NBUF * CHUNK
    n_rounds = -(-n_edges // round_edges)
    if n_rounds % 2:
        n_rounds += 1
    n_pad = n_rounds * round_edges - n_edges
    if n_pad:
        lane = jnp.arange(n_pad, dtype=jnp.int32)
        pad = jnp.stack([lane % n_nodes,
                         n_nodes + lane % ACC_PAD])
        edge_index = jnp.concatenate([edge_index, pad], axis=1)
    edge5 = edge_index.reshape(2, NUM_WORKERS, n_rounds, NBUF, CHUNK)
    partials = _sc_aggregate(x, edge5)
    batch3d = batch.reshape(2, 1, n_nodes // 2)
    return _tc_head(partials, batch3d,
                    W_enc, b_enc.reshape(1, d), W_proj,
                    b_proj.reshape(1, W_proj.shape[1]), n_graphs)
